# baseline (speedup 1.0000x reference)
import jax
import jax.numpy as jnp
from jax import lax
from jax.experimental import pallas as pl
from jax.experimental.pallas import tpu as pltpu
from jax.experimental.pallas import tpu_sc as plsc

B, L, D = 4, 2048, 1024
SLOTS = 1024
TOP_K = 8
SCALE = D ** (-0.5)
EPS = 1e-5

TL = 512
NL = L // TL
CDIMS = (((1,), (1,)), ((), ()))

_CES = ((0, 1), (2, 3), (4, 5), (6, 7),
        (0, 2), (1, 3), (4, 6), (5, 7),
        (1, 2), (5, 6),
        (0, 4), (1, 5), (2, 6), (3, 7),
        (1, 4), (3, 6),
        (2, 4), (3, 5),
        (3, 4))
_NPARTS = SLOTS // 128

_SCI = plsc.get_sparse_core_info()
_NW = _SCI.num_cores * _SCI.num_subcores
_LN = _SCI.num_lanes
_RPW = SLOTS // _NW
_CHUNKS = D // _LN


def _bf16(a):
    return a.astype(jnp.bfloat16)


def _topk_rows(s):
    parts = [s[:, j * 128:(j + 1) * 128] for j in range(_NPARTS)]
    for i, j in _CES:
        a, b = parts[i], parts[j]
        parts[i] = jnp.maximum(a, b)
        parts[j] = jnp.minimum(a, b)
    m1 = None
    cum = None
    kth = None
    for i in range(TOP_K):
        front = parts[0]
        m = jnp.max(front, axis=1, keepdims=True)
        eq = front == m
        c = jnp.sum(eq.astype(s.dtype), axis=1, keepdims=True)
        if i == 0:
            m1 = m
            kth = m
            cum = c
        else:
            take = jnp.logical_and(cum < TOP_K, cum + c >= TOP_K)
            kth = jnp.where(take, m, kth)
            cum = cum + c
        for j in range(_NPARTS - 1 - i):
            parts[j] = jnp.where(eq, parts[j + 1], parts[j])
        if i < TOP_K - 1:
            parts[_NPARTS - 1 - i] = jnp.where(eq, -jnp.inf,
                                               parts[_NPARTS - 1 - i])
    return kth, m1


def _sparse_softmax_rows(s):
    kth, m1 = _topk_rows(s)
    e = jnp.where(s >= kth, jnp.exp((s - m1) * SCALE), 0.0)
    return e / jnp.sum(e, axis=1, keepdims=True)


def _read_kernel(x_ref, M_ref, Wq_ref, Wo_ref, gate_ref, lnw_ref, lnb_ref,
                 out_ref, rep_ref, M16_ref, Wo16_ref):
    x = x_ref[0]
    Mb = M_ref[0]

    @pl.when(pl.program_id(1) == 0)
    def _casts():
        M16_ref[...] = _bf16(Mb)
        Wo16_ref[...] = _bf16(Wo_ref[...])

    q = jax.lax.dot_general(x, Wq_ref[...], CDIMS,
                            preferred_element_type=jnp.float32)
    s = jax.lax.dot_general(q, Mb, CDIMS,
                            preferred_element_type=jnp.float32)
    addr = _sparse_softmax_rows(s)
    r = jax.lax.dot_general(_bf16(addr), M16_ref[...],
                            (((1,), (0,)), ((), ())),
                            preferred_element_type=jnp.float32)
    mu = jnp.mean(r, axis=1, keepdims=True)
    var = jnp.mean((r - mu) ** 2, axis=1, keepdims=True)
    ln = (r - mu) * jax.lax.rsqrt(var + EPS) * lnw_ref[...] + lnb_ref[...]
    out_ref[0] = jax.lax.dot_general(_bf16(ln), Wo16_ref[...], CDIMS,
                                     preferred_element_type=jnp.float32)
    rep_ref[0] = jax.nn.sigmoid(gate_ref[...]) * r


def _addr_kernel(xl_ref, M_ref, Wq_ref, We_ref, be_ref, Wa_ref, ba_ref,
                 aw_ref, er_ref, ad_ref):
    xl = xl_ref[:, 0, :]
    q = jax.lax.dot_general(xl, Wq_ref[...], CDIMS,
                            preferred_element_type=jnp.float32)
    rows = []
    for b in range(B):
        s_col = jax.lax.dot_general(M_ref[b], q[b:b + 1], CDIMS,
                                    preferred_element_type=jnp.float32)
        rows.append(s_col.T)
    s = jnp.concatenate(rows, axis=0)
    kth, m1 = _topk_rows(s)
    e = jnp.where(s >= kth, jnp.exp((s - m1) * SCALE), 0.0)
    aw_ref[:, 0, :] = e / jnp.sum(e, axis=1, keepdims=True)
    er_ref[:, 0, :] = jax.nn.sigmoid(
        jax.lax.dot_general(xl, We_ref[...], CDIMS,
                            preferred_element_type=jnp.float32) + be_ref[...])
    ad_ref[:, 0, :] = jnp.tanh(
        jax.lax.dot_general(xl, Wa_ref[...], CDIMS,
                            preferred_element_type=jnp.float32) + ba_ref[...])


def _sc_write_kernel(aw_hbm, er_hbm, ad_hbm, M_hbm, Mnew_hbm,
                     aw_v, er_v, ad_v, M_v):
    wid = lax.axis_index("s") * _SCI.num_cores + lax.axis_index("c")
    base = wid * _RPW
    for b in range(B):
        pltpu.sync_copy(aw_hbm.at[b, pl.ds(base, _RPW)],
                        aw_v.at[pl.ds(b * _RPW, _RPW)])
    pltpu.sync_copy(er_hbm, er_v)
    pltpu.sync_copy(ad_hbm, ad_v)
    gd = lax.GatherDimensionNumbers(
        offset_dims=(), collapsed_slice_dims=(0,), start_index_map=(0,))
    for b in range(B):
        pltpu.sync_copy(M_hbm.at[b, pl.ds(base, _RPW)], M_v)
        for g in range(_RPW // _LN):
            awc = aw_v[pl.ds(b * _RPW + g * _LN, _LN)]
            a_regs = [
                lax.gather(awc, jnp.full((_LN, 1), k, jnp.int32), gd, (1,),
                           mode=lax.GatherScatterMode.PROMISE_IN_BOUNDS)
                for k in range(_LN)
            ]

            def chunk_body(c, carry, b=b, g=g, a_regs=a_regs):
                off = c * _LN
                p = er_v[b, pl.ds(off, _LN)]
                u = ad_v[b, pl.ds(off, _LN)]
                for k in range(_LN):
                    row = g * _LN + k
                    m = M_v[row, pl.ds(off, _LN)]
                    M_v[row, pl.ds(off, _LN)] = (
                        m - a_regs[k] * (m * p - u))
                return carry

            lax.fori_loop(0, _CHUNKS, chunk_body, 0)
        pltpu.sync_copy(M_v, Mnew_hbm.at[b, pl.ds(base, _RPW)])


def kernel(x, M, W_q, W_e, b_e, W_a, b_a, W_o, r_gate, ln_w, ln_b):
    gate2 = r_gate.reshape(1, D)
    lnw2 = ln_w.reshape(1, D)
    lnb2 = ln_b.reshape(1, D)
    be2 = b_e.reshape(1, D)
    ba2 = b_a.reshape(1, D)
    x_last = x[:, -1].reshape(B, 1, D)

    aw, er, ad = pl.pallas_call(
        _addr_kernel,
        in_specs=[
            pl.BlockSpec((B, 1, D), lambda: (0, 0, 0)),
            pl.BlockSpec((B, SLOTS, D), lambda: (0, 0, 0)),
            pl.BlockSpec((D, D), lambda: (0, 0)),
            pl.BlockSpec((D, D), lambda: (0, 0)),
            pl.BlockSpec((1, D), lambda: (0, 0)),
            pl.BlockSpec((D, D), lambda: (0, 0)),
            pl.BlockSpec((1, D), lambda: (0, 0)),
        ],
        out_specs=[
            pl.BlockSpec((B, 1, SLOTS), lambda: (0, 0, 0)),
            pl.BlockSpec((B, 1, D), lambda: (0, 0, 0)),
            pl.BlockSpec((B, 1, D), lambda: (0, 0, 0)),
        ],
        out_shape=[
            jax.ShapeDtypeStruct((B, 1, SLOTS), jnp.float32),
            jax.ShapeDtypeStruct((B, 1, D), jnp.float32),
            jax.ShapeDtypeStruct((B, 1, D), jnp.float32),
        ],
    )(x_last, M, W_q, W_e, be2, W_a, ba2)

    wcell = lambda b, l: (0, 0)
    out, rep = pl.pallas_call(
        _read_kernel,
        grid=(B, NL),
        in_specs=[
            pl.BlockSpec((1, TL, D), lambda b, l: (b, l, 0)),
            pl.BlockSpec((1, SLOTS, D), lambda b, l: (b, 0, 0)),
            pl.BlockSpec((D, D), wcell),
            pl.BlockSpec((D, D), wcell),
            pl.BlockSpec((1, D), wcell),
            pl.BlockSpec((1, D), wcell),
            pl.BlockSpec((1, D), wcell),
        ],
        out_specs=[
            pl.BlockSpec((1, TL, D), lambda b, l: (b, l, 0)),
            pl.BlockSpec((1, TL, D), lambda b, l: (b, l, 0)),
        ],
        out_shape=[
            jax.ShapeDtypeStruct((B, L, D), jnp.float32),
            jax.ShapeDtypeStruct((B, L, D), jnp.float32),
        ],
        scratch_shapes=[
            pltpu.VMEM((SLOTS, D), jnp.bfloat16),
            pltpu.VMEM((D, D), jnp.bfloat16),
        ],
    )(x, M, W_q, W_o, gate2, lnw2, lnb2)

    mesh = plsc.VectorSubcoreMesh(core_axis_name="c", subcore_axis_name="s")
    sc_write = pl.kernel(
        _sc_write_kernel,
        mesh=mesh,
        out_type=jax.ShapeDtypeStruct((B, SLOTS, D), jnp.float32),
        scratch_types=[
            pltpu.VMEM((B * _RPW,), jnp.float32),
            pltpu.VMEM((B, D), jnp.float32),
            pltpu.VMEM((B, D), jnp.float32),
            pltpu.VMEM((_RPW, D), jnp.float32),
        ],
    )
    M_new = sc_write(aw.reshape(B, SLOTS), er.reshape(B, D),
                     ad.reshape(B, D), M)

    return out, rep, M_new


# speedup vs baseline: 1.0953x; 1.0953x over previous
import jax
import jax.numpy as jnp
from jax import lax
from jax.experimental import pallas as pl
from jax.experimental.pallas import tpu as pltpu
from jax.experimental.pallas import tpu_sc as plsc

B, L, D = 4, 2048, 1024
SLOTS = 1024
TOP_K = 8
SCALE = D ** (-0.5)
EPS = 1e-5

TL = 1024
NL = L // TL
CDIMS = (((1,), (1,)), ((), ()))

_CES = ((0, 1), (2, 3), (4, 5), (6, 7),
        (0, 2), (1, 3), (4, 6), (5, 7),
        (1, 2), (5, 6),
        (0, 4), (1, 5), (2, 6), (3, 7),
        (1, 4), (3, 6),
        (2, 4), (3, 5),
        (3, 4))
_NPARTS = SLOTS // 128

_SCI = plsc.get_sparse_core_info()
_NW = _SCI.num_cores * _SCI.num_subcores
_LN = _SCI.num_lanes
_RPW = SLOTS // _NW
_CHUNKS = D // _LN


def _bf16(a):
    return a.astype(jnp.bfloat16)


def _topk_rows(s):
    parts = [s[:, j * 128:(j + 1) * 128] for j in range(_NPARTS)]
    for i, j in _CES:
        a, b = parts[i], parts[j]
        parts[i] = jnp.maximum(a, b)
        parts[j] = jnp.minimum(a, b)
    m1 = None
    cum = None
    kth = None
    for i in range(TOP_K):
        front = parts[0]
        m = jnp.max(front, axis=1, keepdims=True)
        eq = front == m
        c = jnp.sum(eq.astype(s.dtype), axis=1, keepdims=True)
        if i == 0:
            m1 = m
            kth = m
            cum = c
        else:
            take = jnp.logical_and(cum < TOP_K, cum + c >= TOP_K)
            kth = jnp.where(take, m, kth)
            cum = cum + c
        for j in range(_NPARTS - 1 - i):
            parts[j] = jnp.where(eq, parts[j + 1], parts[j])
        if i < TOP_K - 1:
            parts[_NPARTS - 1 - i] = jnp.where(eq, -jnp.inf,
                                               parts[_NPARTS - 1 - i])
    return kth, m1


def _sparse_softmax_rows(s):
    kth, m1 = _topk_rows(s)
    e = jnp.where(s >= kth, jnp.exp((s - m1) * SCALE), 0.0)
    return e / jnp.sum(e, axis=1, keepdims=True)


def _read_kernel(x_ref, M_ref, Wq_ref, Wo_ref, gate_ref, lnw_ref, lnb_ref,
                 out_ref, rep_ref):
    x = x_ref[0]
    Mb = M_ref[0]
    q = jax.lax.dot_general(x, Wq_ref[...], CDIMS,
                            preferred_element_type=jnp.float32)
    s = jax.lax.dot_general(q, Mb, CDIMS,
                            preferred_element_type=jnp.float32)
    addr = _sparse_softmax_rows(s)
    r = jax.lax.dot_general(_bf16(addr), _bf16(Mb), (((1,), (0,)), ((), ())),
                            preferred_element_type=jnp.float32)
    mu = jnp.mean(r, axis=1, keepdims=True)
    var = jnp.mean((r - mu) ** 2, axis=1, keepdims=True)
    ln = (r - mu) * jax.lax.rsqrt(var + EPS) * lnw_ref[...] + lnb_ref[...]
    out_ref[0] = jax.lax.dot_general(_bf16(ln), _bf16(Wo_ref[...]), CDIMS,
                                     preferred_element_type=jnp.float32)
    rep_ref[0] = jax.nn.sigmoid(gate_ref[...]) * r


def _addr_kernel(xl_ref, M_ref, Wq_ref, We_ref, be_ref, Wa_ref, ba_ref,
                 aw_ref, er_ref, ad_ref):
    xl = xl_ref[:, 0, :]
    q = jax.lax.dot_general(xl, Wq_ref[...], CDIMS,
                            preferred_element_type=jnp.float32)
    rows = []
    for b in range(B):
        s_col = jax.lax.dot_general(M_ref[b], q[b:b + 1], CDIMS,
                                    preferred_element_type=jnp.float32)
        rows.append(s_col.T)
    s = jnp.concatenate(rows, axis=0)
    kth, m1 = _topk_rows(s)
    e = jnp.where(s >= kth, jnp.exp((s - m1) * SCALE), 0.0)
    aw_ref[:, 0, :] = e / jnp.sum(e, axis=1, keepdims=True)
    er_ref[:, 0, :] = jax.nn.sigmoid(
        jax.lax.dot_general(xl, We_ref[...], CDIMS,
                            preferred_element_type=jnp.float32) + be_ref[...])
    ad_ref[:, 0, :] = jnp.tanh(
        jax.lax.dot_general(xl, Wa_ref[...], CDIMS,
                            preferred_element_type=jnp.float32) + ba_ref[...])


def _sc_write_kernel(aw_hbm, er_hbm, ad_hbm, M_hbm, Mnew_hbm,
                     aw_v, er_v, ad_v, M_v):
    wid = lax.axis_index("s") * _SCI.num_cores + lax.axis_index("c")
    base = wid * _RPW
    for b in range(B):
        pltpu.sync_copy(aw_hbm.at[b, pl.ds(base, _RPW)],
                        aw_v.at[pl.ds(b * _RPW, _RPW)])
    pltpu.sync_copy(er_hbm, er_v)
    pltpu.sync_copy(ad_hbm, ad_v)
    gd = lax.GatherDimensionNumbers(
        offset_dims=(), collapsed_slice_dims=(0,), start_index_map=(0,))
    for b in range(B):
        pltpu.sync_copy(M_hbm.at[b, pl.ds(base, _RPW)], M_v)
        for g in range(_RPW // _LN):
            awc = aw_v[pl.ds(b * _RPW + g * _LN, _LN)]
            a_regs = [
                lax.gather(awc, jnp.full((_LN, 1), k, jnp.int32), gd, (1,),
                           mode=lax.GatherScatterMode.PROMISE_IN_BOUNDS)
                for k in range(_LN)
            ]

            def chunk_body(c, carry, b=b, g=g, a_regs=a_regs):
                off = c * _LN
                p = er_v[b, pl.ds(off, _LN)]
                u = ad_v[b, pl.ds(off, _LN)]
                for k in range(_LN):
                    row = g * _LN + k
                    m = M_v[row, pl.ds(off, _LN)]
                    M_v[row, pl.ds(off, _LN)] = (
                        m - a_regs[k] * (m * p - u))
                return carry

            lax.fori_loop(0, _CHUNKS, chunk_body, 0)
        pltpu.sync_copy(M_v, Mnew_hbm.at[b, pl.ds(base, _RPW)])


def kernel(x, M, W_q, W_e, b_e, W_a, b_a, W_o, r_gate, ln_w, ln_b):
    gate2 = r_gate.reshape(1, D)
    lnw2 = ln_w.reshape(1, D)
    lnb2 = ln_b.reshape(1, D)
    be2 = b_e.reshape(1, D)
    ba2 = b_a.reshape(1, D)
    x_last = x[:, -1].reshape(B, 1, D)

    aw, er, ad = pl.pallas_call(
        _addr_kernel,
        in_specs=[
            pl.BlockSpec((B, 1, D), lambda: (0, 0, 0)),
            pl.BlockSpec((B, SLOTS, D), lambda: (0, 0, 0)),
            pl.BlockSpec((D, D), lambda: (0, 0)),
            pl.BlockSpec((D, D), lambda: (0, 0)),
            pl.BlockSpec((1, D), lambda: (0, 0)),
            pl.BlockSpec((D, D), lambda: (0, 0)),
            pl.BlockSpec((1, D), lambda: (0, 0)),
        ],
        out_specs=[
            pl.BlockSpec((B, 1, SLOTS), lambda: (0, 0, 0)),
            pl.BlockSpec((B, 1, D), lambda: (0, 0, 0)),
            pl.BlockSpec((B, 1, D), lambda: (0, 0, 0)),
        ],
        out_shape=[
            jax.ShapeDtypeStruct((B, 1, SLOTS), jnp.float32),
            jax.ShapeDtypeStruct((B, 1, D), jnp.float32),
            jax.ShapeDtypeStruct((B, 1, D), jnp.float32),
        ],
    )(x_last, M, W_q, W_e, be2, W_a, ba2)

    wcell = lambda b, l: (0, 0)
    out, rep = pl.pallas_call(
        _read_kernel,
        grid=(B, NL),
        in_specs=[
            pl.BlockSpec((1, TL, D), lambda b, l: (b, l, 0)),
            pl.BlockSpec((1, SLOTS, D), lambda b, l: (b, 0, 0)),
            pl.BlockSpec((D, D), wcell),
            pl.BlockSpec((D, D), wcell),
            pl.BlockSpec((1, D), wcell),
            pl.BlockSpec((1, D), wcell),
            pl.BlockSpec((1, D), wcell),
        ],
        out_specs=[
            pl.BlockSpec((1, TL, D), lambda b, l: (b, l, 0)),
            pl.BlockSpec((1, TL, D), lambda b, l: (b, l, 0)),
        ],
        out_shape=[
            jax.ShapeDtypeStruct((B, L, D), jnp.float32),
            jax.ShapeDtypeStruct((B, L, D), jnp.float32),
        ],
    )(x, M, W_q, W_o, gate2, lnw2, lnb2)

    mesh = plsc.VectorSubcoreMesh(core_axis_name="c", subcore_axis_name="s")
    sc_write = pl.kernel(
        _sc_write_kernel,
        mesh=mesh,
        out_type=jax.ShapeDtypeStruct((B, SLOTS, D), jnp.float32),
        scratch_types=[
            pltpu.VMEM((B * _RPW,), jnp.float32),
            pltpu.VMEM((B, D), jnp.float32),
            pltpu.VMEM((B, D), jnp.float32),
            pltpu.VMEM((_RPW, D), jnp.float32),
        ],
    )
    M_new = sc_write(aw.reshape(B, SLOTS), er.reshape(B, D),
                     ad.reshape(B, D), M)

    return out, rep, M_new
